# Initial kernel scaffold; baseline (speedup 1.0000x reference)
import jax
import jax.numpy as jnp
from jax.experimental import pallas as pl


def kernel(x, edge_index, W1_l, W1_r, b1, W2_l, W2_r, b2):
    raise NotImplementedError("write your pallas kernel here")


# speedup vs baseline: 8.5919x; 8.5919x over previous
import functools

import jax
import jax.numpy as jnp
from jax import lax
from jax.experimental import pallas as pl
from jax.experimental.pallas import tpu as pltpu
from jax.experimental.pallas import tpu_sc as plsc

N = 10000
E = 160000
D_IN = 256
D_HID = 128
DH2 = D_HID // 2
N_CLS = 2

NC = 2
NS = 16
CH = 128
K2_KCH = 79
K2_EPT = K2_KCH * CH
K2_SPLIT = 40
K4_KCH = 40
R = 10240
ZR = 64
RPS = R // NS
BR = 2000


def _mm1_body(x_ref, wl_ref, wr_ref, y_ref, r1_ref):
    xb = x_ref[...]
    y_ref[0] = jnp.dot(xb, wl_ref[:, :DH2], preferred_element_type=jnp.float32)
    y_ref[1] = jnp.dot(xb, wl_ref[:, DH2:], preferred_element_type=jnp.float32)
    r1_ref[...] = jnp.dot(xb, wr_ref[...], preferred_element_type=jnp.float32)


def _k3_body(aggp_ref, cntp_ref, r1_ref, b1_ref, w2_ref, u_ref):
    a = jnp.concatenate([aggp_ref[0], aggp_ref[1]], axis=1)
    cnt = cntp_ref[0, :, 0] + cntp_ref[1, :, 0]
    inv = 1.0 / jnp.maximum(cnt, 1.0)
    h = jnp.maximum(a * inv[:, None] + r1_ref[...] + b1_ref[...], 0.0)
    u_ref[...] = jnp.dot(h, w2_ref[...], preferred_element_type=jnp.float32)


def _k5_body(a2p_ref, cntp_ref, u_ref, b2_ref, o_ref):
    a = a2p_ref[0] + a2p_ref[1]
    cnt = cntp_ref[0, :, 0] + cntp_ref[1, :, 0]
    inv = 1.0 / jnp.maximum(cnt, 1.0)
    o_ref[...] = a[:, 0:2] * inv[:, None] + u_ref[:, 2:4] + b2_ref[...]


_MESH = plsc.VectorSubcoreMesh(core_axis_name="c", subcore_axis_name="s")


def _sc_scatter1(ys_h, src_h, dst_h, ones_h, zf_h, z16_h, aggp_h, cntp_h,
                 src_v, dst_v, buf, ones_v, stage16, acc_s, cnt_s):
    c = lax.axis_index("c")
    s = lax.axis_index("s")
    pltpu.sync_copy(zf_h, buf)
    pltpu.sync_copy(z16_h, stage16)
    pltpu.sync_copy(ones_h, ones_v)
    pltpu.sync_copy(src_h.at[c, s], src_v)
    pltpu.sync_copy(dst_h.at[s], dst_v)
    base = s * RPS

    def zbody(i, carry):
        pltpu.sync_copy(buf.at[pl.ds(0, ZR)], acc_s.at[pl.ds(base + i * ZR, ZR)])
        pltpu.sync_copy(stage16, cnt_s.at[pl.ds(base + i * ZR, ZR)])
        return carry

    lax.fori_loop(0, RPS // ZR, zbody, 0)
    plsc.subcore_barrier()

    def ebody(j, carry):
        pltpu.sync_copy(ys_h.at[src_v.at[j]], buf)
        pltpu.sync_copy(buf, acc_s.at[dst_v.at[j]], add=True)
        count_here = jnp.logical_xor(c == 1, j < K2_SPLIT)

        @pl.when(count_here)
        def _():
            pltpu.sync_copy(ones_v, cnt_s.at[dst_v.at[j]], add=True)

        return carry

    lax.fori_loop(0, K2_KCH, ebody, 0)
    plsc.subcore_barrier()

    def dbody(i, carry):
        rows = pl.ds(base + i * ZR, ZR)
        pltpu.sync_copy(acc_s.at[rows], buf.at[pl.ds(0, ZR)])
        pltpu.sync_copy(buf.at[pl.ds(0, ZR)], aggp_h.at[c, rows])
        pltpu.sync_copy(cnt_s.at[rows], stage16)
        pltpu.sync_copy(stage16, cntp_h.at[c, rows])
        return carry

    lax.fori_loop(0, RPS // ZR, dbody, 0)


def _sc_scatter2(u_h, src_h, dst_h, z16_h, a2p_h,
                 src_v, dst_v, buf, stage16, acc_s):
    c = lax.axis_index("c")
    s = lax.axis_index("s")
    pltpu.sync_copy(z16_h, stage16)
    pltpu.sync_copy(src_h.at[c, s], src_v)
    pltpu.sync_copy(dst_h.at[c, s], dst_v)
    base = s * RPS

    def zbody(i, carry):
        pltpu.sync_copy(stage16, acc_s.at[pl.ds(base + i * ZR, ZR)])
        return carry

    lax.fori_loop(0, RPS // ZR, zbody, 0)
    plsc.subcore_barrier()

    def ebody(j, carry):
        pltpu.sync_copy(u_h.at[src_v.at[j]], buf)
        pltpu.sync_copy(buf, acc_s.at[dst_v.at[j]], add=True)
        return carry

    lax.fori_loop(0, K4_KCH, ebody, 0)
    plsc.subcore_barrier()

    def dbody(i, carry):
        rows = pl.ds(base + i * ZR, ZR)
        pltpu.sync_copy(acc_s.at[rows], stage16)
        pltpu.sync_copy(stage16, a2p_h.at[c, rows])
        return carry

    lax.fori_loop(0, RPS // ZR, dbody, 0)


_k2_call = functools.partial(
    pl.kernel,
    _sc_scatter1,
    out_type=(
        jax.ShapeDtypeStruct((NC, R, DH2), jnp.float32),
        jax.ShapeDtypeStruct((NC, R, 16), jnp.float32),
    ),
    mesh=_MESH,
    compiler_params=pltpu.CompilerParams(use_tc_tiling_on_sc=False),
    scratch_types=[
        pltpu.VMEM((K2_KCH, CH), jnp.int32),
        pltpu.VMEM((K2_KCH, CH), jnp.int32),
        pltpu.VMEM((CH, DH2), jnp.float32),
        pltpu.VMEM((CH, 16), jnp.float32),
        pltpu.VMEM((ZR, 16), jnp.float32),
        pltpu.VMEM_SHARED((R, DH2), jnp.float32),
        pltpu.VMEM_SHARED((R, 16), jnp.float32),
    ],
)()

_k4_call = functools.partial(
    pl.kernel,
    _sc_scatter2,
    out_type=jax.ShapeDtypeStruct((NC, R, 16), jnp.float32),
    mesh=_MESH,
    compiler_params=pltpu.CompilerParams(use_tc_tiling_on_sc=False),
    scratch_types=[
        pltpu.VMEM((K4_KCH, CH), jnp.int32),
        pltpu.VMEM((K4_KCH, CH), jnp.int32),
        pltpu.VMEM((CH, 16), jnp.float32),
        pltpu.VMEM((ZR, 16), jnp.float32),
        pltpu.VMEM_SHARED((R, 16), jnp.float32),
    ],
)()


def kernel(x, edge_index, W1_l, W1_r, b1, W2_l, W2_r, b2):
    src = edge_index[0].astype(jnp.int32)
    dst = edge_index[1].astype(jnp.int32)

    npad2 = NS * K2_EPT - E
    ar2 = jnp.arange(npad2, dtype=jnp.int32)
    pad_src2 = (ar2 * 37) % N
    pad_dst2 = N + (ar2 % (R - N))
    src2 = jnp.concatenate([src, pad_src2]).reshape(NS, K2_KCH, CH)
    src2 = jnp.stack([src2, src2 + N])
    dst2 = jnp.concatenate([dst, pad_dst2]).reshape(NS, K2_KCH, CH)

    npad4 = NC * NS * K4_KCH * CH - E
    ar4 = jnp.arange(npad4, dtype=jnp.int32)
    pad_src4 = (ar4 * 37) % N
    pad_dst4 = N + (ar4 % (R - N))
    src4 = jnp.concatenate([src, pad_src4]).reshape(NC, NS, K4_KCH, CH)
    dst4 = jnp.concatenate([dst, pad_dst4]).reshape(NC, NS, K4_KCH, CH)

    ones16 = jnp.ones((CH, 16), jnp.float32)
    zf = jnp.zeros((CH, DH2), jnp.float32)
    z16 = jnp.zeros((ZR, 16), jnp.float32)
    w2 = jnp.zeros((D_HID, 16), jnp.float32)
    w2 = w2.at[:, 0:2].set(W2_l).at[:, 2:4].set(W2_r)

    ystk, r1 = pl.pallas_call(
        _mm1_body,
        grid=(N // BR,),
        in_specs=[
            pl.BlockSpec((BR, D_IN), lambda i: (i, 0)),
            pl.BlockSpec((D_IN, D_HID), lambda i: (0, 0)),
            pl.BlockSpec((D_IN, D_HID), lambda i: (0, 0)),
        ],
        out_specs=[
            pl.BlockSpec((NC, BR, DH2), lambda i: (0, i, 0)),
            pl.BlockSpec((BR, D_HID), lambda i: (i, 0)),
        ],
        out_shape=[
            jax.ShapeDtypeStruct((NC, N, DH2), jnp.float32),
            jax.ShapeDtypeStruct((N, D_HID), jnp.float32),
        ],
    )(x, W1_l, W1_r)
    ys = ystk.reshape(NC * N, DH2)

    aggp, cntp = _k2_call(ys, src2, dst2, ones16, zf, z16)

    u = pl.pallas_call(
        _k3_body,
        grid=(N // BR,),
        in_specs=[
            pl.BlockSpec((NC, BR, DH2), lambda i: (0, i, 0)),
            pl.BlockSpec((NC, BR, 16), lambda i: (0, i, 0)),
            pl.BlockSpec((BR, D_HID), lambda i: (i, 0)),
            pl.BlockSpec((1, D_HID), lambda i: (0, 0)),
            pl.BlockSpec((D_HID, 16), lambda i: (0, 0)),
        ],
        out_specs=pl.BlockSpec((BR, 16), lambda i: (i, 0)),
        out_shape=jax.ShapeDtypeStruct((N, 16), jnp.float32),
    )(aggp, cntp, r1, b1.reshape(1, D_HID), w2)

    a2p = _k4_call(u, src4, dst4, z16)

    out = pl.pallas_call(
        _k5_body,
        grid=(N // BR,),
        in_specs=[
            pl.BlockSpec((NC, BR, 16), lambda i: (0, i, 0)),
            pl.BlockSpec((NC, BR, 16), lambda i: (0, i, 0)),
            pl.BlockSpec((BR, 16), lambda i: (i, 0)),
            pl.BlockSpec((1, N_CLS), lambda i: (0, 0)),
        ],
        out_specs=pl.BlockSpec((BR, N_CLS), lambda i: (i, 0)),
        out_shape=jax.ShapeDtypeStruct((N, N_CLS), jnp.float32),
    )(a2p, cntp, u, b2.reshape(1, N_CLS))

    return out


# speedup vs baseline: 12.4325x; 1.4470x over previous
import functools

import jax
import jax.numpy as jnp
from jax import lax
from jax.experimental import pallas as pl
from jax.experimental.pallas import tpu as pltpu
from jax.experimental.pallas import tpu_sc as plsc

N = 10000
E = 160000
D_IN = 256
D_HID = 128
DH2 = D_HID // 2
N_CLS = 2

NC = 2
NS = 16
CH = 128
NB = 4
K2_KCH = 80
K2_EPT = K2_KCH * CH
K2_SPLIT = 40
K4_KCH = 40
R = 10240
ZR = 64
RPS = R // NS
ZB = RPS // CH
BR = 2000


def _mm1_body(x_ref, wl_ref, wr_ref, y_ref, r1_ref):
    xb = x_ref[...]
    y_ref[0] = jnp.dot(xb, wl_ref[:, :DH2], preferred_element_type=jnp.float32)
    y_ref[1] = jnp.dot(xb, wl_ref[:, DH2:], preferred_element_type=jnp.float32)
    r1_ref[...] = jnp.dot(xb, wr_ref[...], preferred_element_type=jnp.float32)


def _k3_body(aggp_ref, cntp_ref, r1_ref, b1_ref, w2_ref, u_ref):
    a = jnp.concatenate([aggp_ref[0], aggp_ref[1]], axis=1)
    cnt = cntp_ref[0, :, 0] + cntp_ref[1, :, 0]
    inv = 1.0 / jnp.maximum(cnt, 1.0)
    h = jnp.maximum(a * inv[:, None] + r1_ref[...] + b1_ref[...], 0.0)
    u_ref[...] = jnp.dot(h, w2_ref[...], preferred_element_type=jnp.float32)


def _k5_body(a2p_ref, cntp_ref, u_ref, b2_ref, o_ref):
    a = a2p_ref[0] + a2p_ref[1]
    cnt = cntp_ref[0, :, 0] + cntp_ref[1, :, 0]
    inv = 1.0 / jnp.maximum(cnt, 1.0)
    o_ref[...] = a[:, 0:2] * inv[:, None] + u_ref[:, 2:4] + b2_ref[...]


_MESH = plsc.VectorSubcoreMesh(core_axis_name="c", subcore_axis_name="s")


def _pipelined_edge_loop(table_h, src_v, dst_v, bufs, sgs, sss, acc_s, kch,
                         extra=None):
    for b in range(NB):
        pltpu.make_async_copy(table_h.at[src_v.at[b]], bufs[b], sgs[b]).start()

    def ebody(jj, carry):
        for b in range(NB):
            j = jj * NB + b
            pb = (b - 1) % NB
            pltpu.make_async_copy(table_h.at[src_v.at[j]], bufs[b], sgs[b]).wait()
            pltpu.make_async_copy(bufs[b], acc_s.at[dst_v.at[j]], sss[b]).start(add=True)
            if extra is not None:
                extra(j)
            nj = j - 1 + NB

            @pl.when(jnp.logical_and(j >= 1, nj < kch))
            def _():
                pltpu.make_async_copy(bufs[pb], acc_s.at[dst_v.at[j]], sss[pb]).wait()
                pltpu.make_async_copy(table_h.at[src_v.at[nj]], bufs[pb], sgs[pb]).start()

        return carry

    lax.fori_loop(0, kch // NB, ebody, 0)
    for b in range(NB):
        pltpu.make_async_copy(bufs[b], acc_s.at[dst_v.at[0]], sss[b]).wait()


def _sc_scatter1(ys_h, src_h, dst_h, ones_h, zf_h, z16_h, aggp_h, cntp_h,
                 src_v, dst_v, b0, b1, b2, b3, ones_v, stage16, acc_s, cnt_s,
                 sg0, sg1, sg2, sg3, ss0, ss1, ss2, ss3, sc):
    bufs = (b0, b1, b2, b3)
    sgs = (sg0, sg1, sg2, sg3)
    sss = (ss0, ss1, ss2, ss3)
    c = lax.axis_index("c")
    s = lax.axis_index("s")
    pltpu.sync_copy(zf_h, b0)
    pltpu.sync_copy(z16_h, stage16)
    pltpu.sync_copy(ones_h, ones_v)
    pltpu.sync_copy(src_h.at[c, s], src_v)
    pltpu.sync_copy(dst_h.at[s], dst_v)
    base = s * RPS

    def zs(i, carry):
        pltpu.make_async_copy(b0, acc_s.at[pl.ds(base + i * CH, CH)], ss0).start()
        pltpu.make_async_copy(
            stage16, cnt_s.at[pl.ds(base + i * CH, CH // 2)], ss1).start()
        pltpu.make_async_copy(
            stage16, cnt_s.at[pl.ds(base + i * CH + CH // 2, CH // 2)], ss1).start()
        return carry

    lax.fori_loop(0, ZB, zs, 0)

    def zw(i, carry):
        pltpu.make_async_copy(b0, acc_s.at[pl.ds(base, CH)], ss0).wait()
        pltpu.make_async_copy(stage16, cnt_s.at[pl.ds(base, ZR)], ss1).wait()
        pltpu.make_async_copy(stage16, cnt_s.at[pl.ds(base, ZR)], ss1).wait()
        return carry

    lax.fori_loop(0, ZB, zw, 0)
    plsc.subcore_barrier()

    def counts(j):
        @pl.when(j < K2_SPLIT)
        def _():
            pltpu.make_async_copy(
                ones_v, cnt_s.at[dst_v.at[c * K2_SPLIT + j]], sc).start(add=True)

    _pipelined_edge_loop(ys_h, src_v, dst_v, bufs, sgs, sss, acc_s, K2_KCH,
                         extra=counts)

    def cdrain(i, carry):
        pltpu.make_async_copy(ones_v, cnt_s.at[dst_v.at[0]], sc).wait()
        return carry

    lax.fori_loop(0, K2_SPLIT, cdrain, 0)
    plsc.subcore_barrier()

    def dbody(i, carry):
        rows = pl.ds(base + i * CH, CH)
        pltpu.sync_copy(acc_s.at[rows], b0)
        pltpu.sync_copy(b0, aggp_h.at[c, rows])
        return carry

    lax.fori_loop(0, ZB, dbody, 0)

    def dbody16(i, carry):
        rows = pl.ds(base + i * ZR, ZR)
        pltpu.sync_copy(cnt_s.at[rows], stage16)
        pltpu.sync_copy(stage16, cntp_h.at[c, rows])
        return carry

    lax.fori_loop(0, RPS // ZR, dbody16, 0)


def _sc_scatter2(u_h, src_h, dst_h, z16_h, a2p_h,
                 src_v, dst_v, b0, b1, b2, b3, stage16, acc_s,
                 sg0, sg1, sg2, sg3, ss0, ss1, ss2, ss3):
    bufs = (b0, b1, b2, b3)
    sgs = (sg0, sg1, sg2, sg3)
    sss = (ss0, ss1, ss2, ss3)
    c = lax.axis_index("c")
    s = lax.axis_index("s")
    pltpu.sync_copy(z16_h, stage16)
    pltpu.sync_copy(src_h.at[c, s], src_v)
    pltpu.sync_copy(dst_h.at[c, s], dst_v)
    base = s * RPS

    def zs(i, carry):
        pltpu.make_async_copy(stage16, acc_s.at[pl.ds(base + i * ZR, ZR)], ss0).start()
        return carry

    lax.fori_loop(0, RPS // ZR, zs, 0)

    def zw(i, carry):
        pltpu.make_async_copy(stage16, acc_s.at[pl.ds(base, ZR)], ss0).wait()
        return carry

    lax.fori_loop(0, RPS // ZR, zw, 0)
    plsc.subcore_barrier()

    _pipelined_edge_loop(u_h, src_v, dst_v, bufs, sgs, sss, acc_s, K4_KCH)
    plsc.subcore_barrier()

    def dbody(i, carry):
        rows = pl.ds(base + i * ZR, ZR)
        pltpu.sync_copy(acc_s.at[rows], stage16)
        pltpu.sync_copy(stage16, a2p_h.at[c, rows])
        return carry

    lax.fori_loop(0, RPS // ZR, dbody, 0)


_k2_call = functools.partial(
    pl.kernel,
    _sc_scatter1,
    out_type=(
        jax.ShapeDtypeStruct((NC, R, DH2), jnp.float32),
        jax.ShapeDtypeStruct((NC, R, 16), jnp.float32),
    ),
    mesh=_MESH,
    compiler_params=pltpu.CompilerParams(use_tc_tiling_on_sc=False),
    scratch_types=[
        pltpu.VMEM((K2_KCH, CH), jnp.int32),
        pltpu.VMEM((K2_KCH, CH), jnp.int32),
        pltpu.VMEM((CH, DH2), jnp.float32),
        pltpu.VMEM((CH, DH2), jnp.float32),
        pltpu.VMEM((CH, DH2), jnp.float32),
        pltpu.VMEM((CH, DH2), jnp.float32),
        pltpu.VMEM((CH, 16), jnp.float32),
        pltpu.VMEM((ZR, 16), jnp.float32),
        pltpu.VMEM_SHARED((R, DH2), jnp.float32),
        pltpu.VMEM_SHARED((R, 16), jnp.float32),
    ] + [pltpu.SemaphoreType.DMA] * 9,
)()

_k4_call = functools.partial(
    pl.kernel,
    _sc_scatter2,
    out_type=jax.ShapeDtypeStruct((NC, R, 16), jnp.float32),
    mesh=_MESH,
    compiler_params=pltpu.CompilerParams(use_tc_tiling_on_sc=False),
    scratch_types=[
        pltpu.VMEM((K4_KCH, CH), jnp.int32),
        pltpu.VMEM((K4_KCH, CH), jnp.int32),
        pltpu.VMEM((CH, 16), jnp.float32),
        pltpu.VMEM((CH, 16), jnp.float32),
        pltpu.VMEM((CH, 16), jnp.float32),
        pltpu.VMEM((CH, 16), jnp.float32),
        pltpu.VMEM((ZR, 16), jnp.float32),
        pltpu.VMEM_SHARED((R, 16), jnp.float32),
    ] + [pltpu.SemaphoreType.DMA] * 8,
)()


def kernel(x, edge_index, W1_l, W1_r, b1, W2_l, W2_r, b2):
    src = edge_index[0].astype(jnp.int32)
    dst = edge_index[1].astype(jnp.int32)

    npad2 = NS * K2_EPT - E
    ar2 = jnp.arange(npad2, dtype=jnp.int32)
    pad_src2 = (ar2 * 37) % N
    pad_dst2 = N + (ar2 % (R - N))
    src2 = jnp.concatenate([src, pad_src2]).reshape(NS, K2_KCH, CH)
    src2 = jnp.stack([src2, src2 + N])
    dst2 = jnp.concatenate([dst, pad_dst2]).reshape(NS, K2_KCH, CH)

    npad4 = NC * NS * K4_KCH * CH - E
    ar4 = jnp.arange(npad4, dtype=jnp.int32)
    pad_src4 = (ar4 * 37) % N
    pad_dst4 = N + (ar4 % (R - N))
    src4 = jnp.concatenate([src, pad_src4]).reshape(NC, NS, K4_KCH, CH)
    dst4 = jnp.concatenate([dst, pad_dst4]).reshape(NC, NS, K4_KCH, CH)

    ones16 = jnp.ones((CH, 16), jnp.float32)
    zf = jnp.zeros((CH, DH2), jnp.float32)
    z16 = jnp.zeros((ZR, 16), jnp.float32)
    w2 = jnp.zeros((D_HID, 16), jnp.float32)
    w2 = w2.at[:, 0:2].set(W2_l).at[:, 2:4].set(W2_r)

    ystk, r1 = pl.pallas_call(
        _mm1_body,
        grid=(N // BR,),
        in_specs=[
            pl.BlockSpec((BR, D_IN), lambda i: (i, 0)),
            pl.BlockSpec((D_IN, D_HID), lambda i: (0, 0)),
            pl.BlockSpec((D_IN, D_HID), lambda i: (0, 0)),
        ],
        out_specs=[
            pl.BlockSpec((NC, BR, DH2), lambda i: (0, i, 0)),
            pl.BlockSpec((BR, D_HID), lambda i: (i, 0)),
        ],
        out_shape=[
            jax.ShapeDtypeStruct((NC, N, DH2), jnp.float32),
            jax.ShapeDtypeStruct((N, D_HID), jnp.float32),
        ],
    )(x, W1_l, W1_r)
    ys = ystk.reshape(NC * N, DH2)

    aggp, cntp = _k2_call(ys, src2, dst2, ones16, zf, z16)

    u = pl.pallas_call(
        _k3_body,
        grid=(N // BR,),
        in_specs=[
            pl.BlockSpec((NC, BR, DH2), lambda i: (0, i, 0)),
            pl.BlockSpec((NC, BR, 16), lambda i: (0, i, 0)),
            pl.BlockSpec((BR, D_HID), lambda i: (i, 0)),
            pl.BlockSpec((1, D_HID), lambda i: (0, 0)),
            pl.BlockSpec((D_HID, 16), lambda i: (0, 0)),
        ],
        out_specs=pl.BlockSpec((BR, 16), lambda i: (i, 0)),
        out_shape=jax.ShapeDtypeStruct((N, 16), jnp.float32),
    )(aggp, cntp, r1, b1.reshape(1, D_HID), w2)

    a2p = _k4_call(u, src4, dst4, z16)

    out = pl.pallas_call(
        _k5_body,
        grid=(N // BR,),
        in_specs=[
            pl.BlockSpec((NC, BR, 16), lambda i: (0, i, 0)),
            pl.BlockSpec((NC, BR, 16), lambda i: (0, i, 0)),
            pl.BlockSpec((BR, 16), lambda i: (i, 0)),
            pl.BlockSpec((1, N_CLS), lambda i: (0, 0)),
        ],
        out_specs=pl.BlockSpec((BR, N_CLS), lambda i: (i, 0)),
        out_shape=jax.ShapeDtypeStruct((N, N_CLS), jnp.float32),
    )(a2p, cntp, u, b2.reshape(1, N_CLS))

    return out


# speedup vs baseline: 13.8107x; 1.1109x over previous
import functools

import jax
import jax.numpy as jnp
from jax import lax
from jax.experimental import pallas as pl
from jax.experimental.pallas import tpu as pltpu
from jax.experimental.pallas import tpu_sc as plsc

N = 10000
E = 160000
D_IN = 256
D_HID = 128
DH2 = D_HID // 2
N_CLS = 2

NC = 2
NS = 16
CH = 128
NB = 4
K2_KCH = 80
K2_EPT = K2_KCH * CH
K2_SPLIT = 40
K4_KCH = 40
R = 10240
ZR = 64
RPS = R // NS
ZB = RPS // CH
BR = 2000


def _mm1_body(x_ref, wl_ref, wr_ref, y_ref, r1_ref):
    xb = x_ref[...]
    y_ref[...] = jnp.dot(xb, wl_ref[...], preferred_element_type=jnp.float32)
    r1_ref[...] = jnp.dot(xb, wr_ref[...], preferred_element_type=jnp.float32)


def _k3_body(aggp_ref, cntp_ref, r1_ref, b1_ref, w2_ref, u_ref):
    a = aggp_ref[...]
    cnt = cntp_ref[0, :, 0] + cntp_ref[1, :, 0]
    inv = 1.0 / jnp.maximum(cnt, 1.0)
    h = jnp.maximum(a * inv[:, None] + r1_ref[...] + b1_ref[...], 0.0)
    u_ref[...] = jnp.dot(h, w2_ref[...], preferred_element_type=jnp.float32)


def _k5_body(a2p_ref, cntp_ref, u_ref, b2_ref, o_ref):
    a = a2p_ref[0] + a2p_ref[1]
    cnt = cntp_ref[0, :, 0] + cntp_ref[1, :, 0]
    inv = 1.0 / jnp.maximum(cnt, 1.0)
    o_ref[...] = a[:, 0:2] * inv[:, None] + u_ref[:, 2:4] + b2_ref[...]


_MESH = plsc.VectorSubcoreMesh(core_axis_name="c", subcore_axis_name="s")


def _pipelined_edge_loop(table_h, src_v, dst_v, bufs, sgs, sss, acc_s, kch,
                         extra=None):
    for b in range(NB):
        pltpu.make_async_copy(table_h.at[src_v.at[b]], bufs[b], sgs[b]).start()

    def ebody(jj, carry):
        for b in range(NB):
            j = jj * NB + b
            pb = (b - 1) % NB
            pltpu.make_async_copy(table_h.at[src_v.at[j]], bufs[b], sgs[b]).wait()
            pltpu.make_async_copy(bufs[b], acc_s.at[dst_v.at[j]], sss[b]).start(add=True)
            if extra is not None:
                extra(j)
            nj = j - 1 + NB

            @pl.when(jnp.logical_and(j >= 1, nj < kch))
            def _():
                pltpu.make_async_copy(bufs[pb], acc_s.at[dst_v.at[j]], sss[pb]).wait()
                pltpu.make_async_copy(table_h.at[src_v.at[nj]], bufs[pb], sgs[pb]).start()

        return carry

    lax.fori_loop(0, kch // NB, ebody, 0)
    for b in range(NB):
        pltpu.make_async_copy(bufs[b], acc_s.at[dst_v.at[0]], sss[b]).wait()


def _sc_scatter1(ys_h, src_h, dst_h, ones_h, zf_h, z16_h, aggp_h, cntp_h,
                 src_v, dst_v, b0, b1, b2, b3, ones_v, stage16, acc_s, cnt_s,
                 sg0, sg1, sg2, sg3, ss0, ss1, ss2, ss3, sc):
    bufs = (b0, b1, b2, b3)
    sgs = (sg0, sg1, sg2, sg3)
    sss = (ss0, ss1, ss2, ss3)
    c = lax.axis_index("c")
    s = lax.axis_index("s")
    pltpu.sync_copy(zf_h, b0)
    pltpu.sync_copy(z16_h, stage16)
    pltpu.sync_copy(ones_h, ones_v)
    pltpu.sync_copy(src_h.at[c, s], src_v)
    pltpu.sync_copy(dst_h.at[s], dst_v)
    base = s * RPS

    def zs(i, carry):
        pltpu.make_async_copy(b0, acc_s.at[pl.ds(base + i * CH, CH)], ss0).start()
        pltpu.make_async_copy(
            stage16, cnt_s.at[pl.ds(base + i * CH, CH // 2)], ss1).start()
        pltpu.make_async_copy(
            stage16, cnt_s.at[pl.ds(base + i * CH + CH // 2, CH // 2)], ss1).start()
        return carry

    lax.fori_loop(0, ZB, zs, 0)

    def zw(i, carry):
        pltpu.make_async_copy(b0, acc_s.at[pl.ds(base, CH)], ss0).wait()
        pltpu.make_async_copy(stage16, cnt_s.at[pl.ds(base, ZR)], ss1).wait()
        pltpu.make_async_copy(stage16, cnt_s.at[pl.ds(base, ZR)], ss1).wait()
        return carry

    lax.fori_loop(0, ZB, zw, 0)
    plsc.subcore_barrier()

    def counts(j):
        @pl.when(j < K2_SPLIT)
        def _():
            pltpu.make_async_copy(
                ones_v, cnt_s.at[dst_v.at[c * K2_SPLIT + j]], sc).start(add=True)

    _pipelined_edge_loop(ys_h, src_v, dst_v, bufs, sgs, sss, acc_s, K2_KCH,
                         extra=counts)

    def cdrain(i, carry):
        pltpu.make_async_copy(ones_v, cnt_s.at[dst_v.at[0]], sc).wait()
        return carry

    lax.fori_loop(0, K2_SPLIT, cdrain, 0)
    plsc.subcore_barrier()

    def dbody(i, carry):
        rows = pl.ds(base + i * CH, CH)
        pltpu.sync_copy(acc_s.at[rows], b0)
        pltpu.sync_copy(b0, aggp_h.at[rows, pl.ds(c * DH2, DH2)])
        return carry

    lax.fori_loop(0, ZB, dbody, 0)

    def dbody16(i, carry):
        rows = pl.ds(base + i * ZR, ZR)
        pltpu.sync_copy(cnt_s.at[rows], stage16)
        pltpu.sync_copy(stage16, cntp_h.at[c, rows])
        return carry

    lax.fori_loop(0, RPS // ZR, dbody16, 0)


def _sc_scatter2(u_h, src_h, dst_h, z16_h, a2p_h,
                 src_v, dst_v, b0, b1, b2, b3, stage16, acc_s,
                 sg0, sg1, sg2, sg3, ss0, ss1, ss2, ss3):
    bufs = (b0, b1, b2, b3)
    sgs = (sg0, sg1, sg2, sg3)
    sss = (ss0, ss1, ss2, ss3)
    c = lax.axis_index("c")
    s = lax.axis_index("s")
    pltpu.sync_copy(z16_h, stage16)
    pltpu.sync_copy(src_h.at[c, s], src_v)
    pltpu.sync_copy(dst_h.at[c, s], dst_v)
    base = s * RPS

    def zs(i, carry):
        pltpu.make_async_copy(stage16, acc_s.at[pl.ds(base + i * ZR, ZR)], ss0).start()
        return carry

    lax.fori_loop(0, RPS // ZR, zs, 0)

    def zw(i, carry):
        pltpu.make_async_copy(stage16, acc_s.at[pl.ds(base, ZR)], ss0).wait()
        return carry

    lax.fori_loop(0, RPS // ZR, zw, 0)
    plsc.subcore_barrier()

    _pipelined_edge_loop(u_h, src_v, dst_v, bufs, sgs, sss, acc_s, K4_KCH)
    plsc.subcore_barrier()

    def dbody(i, carry):
        rows = pl.ds(base + i * ZR, ZR)
        pltpu.sync_copy(acc_s.at[rows], stage16)
        pltpu.sync_copy(stage16, a2p_h.at[c, rows])
        return carry

    lax.fori_loop(0, RPS // ZR, dbody, 0)


_k2_call = functools.partial(
    pl.kernel,
    _sc_scatter1,
    out_type=(
        jax.ShapeDtypeStruct((R, D_HID), jnp.float32),
        jax.ShapeDtypeStruct((NC, R, 16), jnp.float32),
    ),
    mesh=_MESH,
    compiler_params=pltpu.CompilerParams(use_tc_tiling_on_sc=False),
    scratch_types=[
        pltpu.VMEM((K2_KCH, CH), jnp.int32),
        pltpu.VMEM((K2_KCH, CH), jnp.int32),
        pltpu.VMEM((CH, DH2), jnp.float32),
        pltpu.VMEM((CH, DH2), jnp.float32),
        pltpu.VMEM((CH, DH2), jnp.float32),
        pltpu.VMEM((CH, DH2), jnp.float32),
        pltpu.VMEM((CH, 16), jnp.float32),
        pltpu.VMEM((ZR, 16), jnp.float32),
        pltpu.VMEM_SHARED((R, DH2), jnp.float32),
        pltpu.VMEM_SHARED((R, 16), jnp.float32),
    ] + [pltpu.SemaphoreType.DMA] * 9,
)()

_k4_call = functools.partial(
    pl.kernel,
    _sc_scatter2,
    out_type=jax.ShapeDtypeStruct((NC, R, 16), jnp.float32),
    mesh=_MESH,
    compiler_params=pltpu.CompilerParams(use_tc_tiling_on_sc=False),
    scratch_types=[
        pltpu.VMEM((K4_KCH, CH), jnp.int32),
        pltpu.VMEM((K4_KCH, CH), jnp.int32),
        pltpu.VMEM((CH, 16), jnp.float32),
        pltpu.VMEM((CH, 16), jnp.float32),
        pltpu.VMEM((CH, 16), jnp.float32),
        pltpu.VMEM((CH, 16), jnp.float32),
        pltpu.VMEM((ZR, 16), jnp.float32),
        pltpu.VMEM_SHARED((R, 16), jnp.float32),
    ] + [pltpu.SemaphoreType.DMA] * 8,
)()


def kernel(x, edge_index, W1_l, W1_r, b1, W2_l, W2_r, b2):
    src = edge_index[0].astype(jnp.int32)
    dst = edge_index[1].astype(jnp.int32)

    npad2 = NS * K2_EPT - E
    ar2 = jnp.arange(npad2, dtype=jnp.int32)
    pad_src2 = (ar2 * 37) % N
    pad_dst2 = N + (ar2 % (R - N))
    src2 = jnp.concatenate([src, pad_src2]).reshape(NS, K2_KCH, CH)
    src2 = jnp.stack([src2 * 2, src2 * 2 + 1])
    dst2 = jnp.concatenate([dst, pad_dst2]).reshape(NS, K2_KCH, CH)

    npad4 = NC * NS * K4_KCH * CH - E
    ar4 = jnp.arange(npad4, dtype=jnp.int32)
    pad_src4 = (ar4 * 37) % N
    pad_dst4 = N + (ar4 % (R - N))
    src4 = jnp.concatenate([src, pad_src4]).reshape(NC, NS, K4_KCH, CH)
    dst4 = jnp.concatenate([dst, pad_dst4]).reshape(NC, NS, K4_KCH, CH)

    ones16 = jnp.ones((CH, 16), jnp.float32)
    zf = jnp.zeros((CH, DH2), jnp.float32)
    z16 = jnp.zeros((ZR, 16), jnp.float32)
    w2 = jnp.zeros((D_HID, 16), jnp.float32)
    w2 = w2.at[:, 0:2].set(W2_l).at[:, 2:4].set(W2_r)

    ystk, r1 = pl.pallas_call(
        _mm1_body,
        grid=(N // BR,),
        in_specs=[
            pl.BlockSpec((BR, D_IN), lambda i: (i, 0)),
            pl.BlockSpec((D_IN, D_HID), lambda i: (0, 0)),
            pl.BlockSpec((D_IN, D_HID), lambda i: (0, 0)),
        ],
        out_specs=[
            pl.BlockSpec((BR, D_HID), lambda i: (i, 0)),
            pl.BlockSpec((BR, D_HID), lambda i: (i, 0)),
        ],
        out_shape=[
            jax.ShapeDtypeStruct((N, D_HID), jnp.float32),
            jax.ShapeDtypeStruct((N, D_HID), jnp.float32),
        ],
    )(x, W1_l, W1_r)
    ys = ystk.reshape(NC * N, DH2)

    aggp, cntp = _k2_call(ys, src2, dst2, ones16, zf, z16)

    u = pl.pallas_call(
        _k3_body,
        grid=(N // BR,),
        in_specs=[
            pl.BlockSpec((BR, D_HID), lambda i: (i, 0)),
            pl.BlockSpec((NC, BR, 16), lambda i: (0, i, 0)),
            pl.BlockSpec((BR, D_HID), lambda i: (i, 0)),
            pl.BlockSpec((1, D_HID), lambda i: (0, 0)),
            pl.BlockSpec((D_HID, 16), lambda i: (0, 0)),
        ],
        out_specs=pl.BlockSpec((BR, 16), lambda i: (i, 0)),
        out_shape=jax.ShapeDtypeStruct((N, 16), jnp.float32),
    )(aggp, cntp, r1, b1.reshape(1, D_HID), w2)

    a2p = _k4_call(u, src4, dst4, z16)

    out = pl.pallas_call(
        _k5_body,
        grid=(1,),
        in_specs=[
            pl.BlockSpec((NC, N, 16), lambda i: (0, 0, 0)),
            pl.BlockSpec((NC, N, 16), lambda i: (0, 0, 0)),
            pl.BlockSpec((N, 16), lambda i: (0, 0)),
            pl.BlockSpec((1, N_CLS), lambda i: (0, 0)),
        ],
        out_specs=pl.BlockSpec((N, N_CLS), lambda i: (0, 0)),
        out_shape=jax.ShapeDtypeStruct((N, N_CLS), jnp.float32),
    )(a2p, cntp, u, b2.reshape(1, N_CLS))

    return out
